# baseline (speedup 1.0000x reference)
import jax
import jax.numpy as jnp
from jax import lax
from jax.experimental import pallas as pl
from jax.experimental.pallas import tpu as pltpu
from jax.experimental.pallas import tpu_sc as plsc

N_LANES = 16
NUM_CORES = 2
NUM_SUBCORES = 16
NUM_WORKERS = NUM_CORES * NUM_SUBCORES
BATCH = 16384
FACTORS = 64
PACKED_W = 2 * FACTORS
ROWS_PER_WORKER = BATCH // NUM_WORKERS
HALF = ROWS_PER_WORKER // 2
GROUPS_PER_HALF = HALF // N_LANES

N_ROWS = 1000000
SPLIT = 512000
REPACK_BN = 12800
REPACK_GRID = SPLIT // REPACK_BN
REPACK_LAST_BLK = (N_ROWS - 1) // REPACK_BN


def _repack_body(lo_ref, hi_ref, out_ref):
    out_ref[:, 0:FACTORS] = lo_ref[...].astype(jnp.bfloat16).T
    out_ref[:, FACTORS:PACKED_W] = hi_ref[...].astype(jnp.bfloat16).T


def _repack(table_t):
    return pl.pallas_call(
        _repack_body,
        out_shape=jax.ShapeDtypeStruct((SPLIT, PACKED_W), jnp.bfloat16),
        grid=(REPACK_GRID,),
        in_specs=[
            pl.BlockSpec((FACTORS, REPACK_BN), lambda i: (0, i)),
            pl.BlockSpec((FACTORS, REPACK_BN),
                         lambda i: (0, jnp.minimum(i + REPACK_GRID,
                                                   REPACK_LAST_BLK))),
        ],
        out_specs=pl.BlockSpec((REPACK_BN, PACKED_W), lambda i: (i, 0)),
        compiler_params=pltpu.CompilerParams(
            fuse_transposed_lhs_in_matmul=True),
    )(table_t, table_t)


def _mf_body(uid2_hbm, iid2_hbm, uoff_hbm, ioff_hbm, pp_hbm, qp_hbm,
             bu_hbm, bi_hbm, uid_hbm, iid_hbm, out_hbm,
             u2_v, i2_v, uoff_v, ioff_v, uidx_v, iidx_v,
             pd_v, qd_v, bu_v, bi_v, out_v, stage_v, sem):
    wid = lax.axis_index("s") * NUM_CORES + lax.axis_index("c")
    base = wid * ROWS_PER_WORKER

    pltpu.sync_copy(uid2_hbm.at[pl.ds(base, ROWS_PER_WORKER)], u2_v)
    pltpu.sync_copy(iid2_hbm.at[pl.ds(base, ROWS_PER_WORKER)], i2_v)
    pltpu.sync_copy(uoff_hbm.at[pl.ds(base, ROWS_PER_WORKER)], uoff_v)
    pltpu.sync_copy(ioff_hbm.at[pl.ds(base, ROWS_PER_WORKER)], ioff_v)
    pltpu.sync_copy(uid_hbm.at[pl.ds(base, ROWS_PER_WORKER)], uidx_v)
    pltpu.sync_copy(iid_hbm.at[pl.ds(base, ROWS_PER_WORKER)], iidx_v)

    cp_bu = pltpu.async_copy(bu_hbm.at[uidx_v], bu_v, sem)
    cp_bi = pltpu.async_copy(bi_hbm.at[iidx_v], bi_v, sem)
    cp_bu.wait()
    cp_bi.wait()

    lane = lax.iota(jnp.int32, N_LANES)

    for h in range(2):
        cp_p = pltpu.async_copy(
            pp_hbm.at[u2_v.at[pl.ds(h * HALF, HALF)]], pd_v, sem)
        cp_q = pltpu.async_copy(
            qp_hbm.at[i2_v.at[pl.ds(h * HALF, HALF)]], qd_v, sem)
        cp_p.wait()
        cp_q.wait()

        def group(g, _):
            rbase0 = h * HALF + g * N_LANES
            uoff_chunk = uoff_v[pl.ds(rbase0, N_LANES)]
            ioff_chunk = ioff_v[pl.ds(rbase0, N_LANES)]
            for l in range(N_LANES):
                r = g * N_LANES + l
                po = uoff_chunk[l]
                qo = ioff_chunk[l]
                v = None
                for j in range(FACTORS // 32):
                    px = pd_v[r, pl.ds(po + j * 32, 32)]
                    qx = qd_v[r, pl.ds(qo + j * 32, 32)]
                    pa, pb = plsc.unpack(px, format=plsc.PackFormat.INTERLEAVED)
                    qa, qb = plsc.unpack(qx, format=plsc.PackFormat.INTERLEAVED)
                    t = pa * qa + pb * qb
                    v = t if v is None else v + t
                stage_v[pl.ds(l * N_LANES, N_LANES)] = v
            rbase = h * HALF + g * N_LANES
            acc = bu_v[pl.ds(rbase, N_LANES)] + bi_v[pl.ds(rbase, N_LANES)]
            for c in range(N_LANES):
                acc = acc + plsc.load_gather(stage_v, [lane * N_LANES + c])
            out_v[pl.ds(rbase, N_LANES)] = acc
            return None

        lax.fori_loop(0, GROUPS_PER_HALF, group, None)

    pltpu.sync_copy(out_v, out_hbm.at[pl.ds(base, ROWS_PER_WORKER)])


@jax.jit
def kernel(user_id, item_id, P, Q, user_bias, item_bias):
    uid = user_id.astype(jnp.int32)
    iid = item_id.astype(jnp.int32)
    mesh = plsc.VectorSubcoreMesh(
        core_axis_name="c", subcore_axis_name="s",
        num_cores=NUM_CORES, num_subcores=NUM_SUBCORES)
    run = pl.kernel(
        _mf_body,
        out_type=jax.ShapeDtypeStruct((BATCH,), jnp.float32),
        mesh=mesh,
        scratch_types=[
            pltpu.VMEM((ROWS_PER_WORKER,), jnp.int32),
            pltpu.VMEM((ROWS_PER_WORKER,), jnp.int32),
            pltpu.VMEM((ROWS_PER_WORKER,), jnp.int32),
            pltpu.VMEM((ROWS_PER_WORKER,), jnp.int32),
            pltpu.VMEM((ROWS_PER_WORKER,), jnp.int32),
            pltpu.VMEM((ROWS_PER_WORKER,), jnp.int32),
            pltpu.VMEM((HALF, PACKED_W), jnp.bfloat16),
            pltpu.VMEM((HALF, PACKED_W), jnp.bfloat16),
            pltpu.VMEM((ROWS_PER_WORKER,), jnp.float32),
            pltpu.VMEM((ROWS_PER_WORKER,), jnp.float32),
            pltpu.VMEM((ROWS_PER_WORKER,), jnp.float32),
            pltpu.VMEM((N_LANES * N_LANES,), jnp.float32),
            pltpu.SemaphoreType.DMA,
        ],
        compiler_params=pltpu.CompilerParams(
            needs_layout_passes=False, use_tc_tiling_on_sc=False),
    )
    u_hi = uid >= SPLIT
    i_hi = iid >= SPLIT
    return run(jnp.where(u_hi, uid - SPLIT, uid),
               jnp.where(i_hi, iid - SPLIT, iid),
               u_hi.astype(jnp.int32) * FACTORS,
               i_hi.astype(jnp.int32) * FACTORS,
               _repack(P.T), _repack(Q.T),
               user_bias.reshape(-1), item_bias.reshape(-1), uid, iid)


# speedup vs baseline: 1.2332x; 1.2332x over previous
import jax
import jax.numpy as jnp
from jax import lax
from jax.experimental import pallas as pl
from jax.experimental.pallas import tpu as pltpu
from jax.experimental.pallas import tpu_sc as plsc

N_LANES = 16
NUM_CORES = 2
NUM_SUBCORES = 16
NUM_WORKERS = NUM_CORES * NUM_SUBCORES
BATCH = 16384
FACTORS = 64
PACKED_W = 2 * FACTORS
ROWS_PER_WORKER = BATCH // NUM_WORKERS
HALF = ROWS_PER_WORKER // 2
GROUPS_PER_HALF = HALF // N_LANES

N_ROWS = 1000000
SPLIT = 512000
REPACK_BN = 12800
REPACK_GRID = SPLIT // REPACK_BN
REPACK_LAST_BLK = (N_ROWS - 1) // REPACK_BN


PACKED_I32 = FACTORS // 2
PACKED_ROW_I32 = 2 * PACKED_I32


def _pack_i32(t):
    a = lax.bitcast_convert_type(t[:, 0:PACKED_I32], jnp.uint16)
    b = lax.bitcast_convert_type(t[:, PACKED_I32:FACTORS], jnp.uint16)
    return a.astype(jnp.int32) | (b.astype(jnp.int32) << 16)


def _repack_body(lo_ref, hi_ref, out_ref):
    out_ref[:, 0:PACKED_I32] = _pack_i32(lo_ref[...].astype(jnp.bfloat16).T)
    out_ref[:, PACKED_I32:PACKED_ROW_I32] = _pack_i32(
        hi_ref[...].astype(jnp.bfloat16).T)


def _repack(table_t):
    return pl.pallas_call(
        _repack_body,
        out_shape=jax.ShapeDtypeStruct((SPLIT, PACKED_ROW_I32), jnp.int32),
        grid=(REPACK_GRID,),
        in_specs=[
            pl.BlockSpec((FACTORS, REPACK_BN), lambda i: (0, i)),
            pl.BlockSpec((FACTORS, REPACK_BN),
                         lambda i: (0, jnp.minimum(i + REPACK_GRID,
                                                   REPACK_LAST_BLK))),
        ],
        out_specs=pl.BlockSpec((REPACK_BN, PACKED_ROW_I32), lambda i: (i, 0)),
        compiler_params=pltpu.CompilerParams(
            fuse_transposed_lhs_in_matmul=True),
    )(table_t, table_t)


def _mf_body(uid2_hbm, iid2_hbm, uoff_hbm, ioff_hbm, pp_hbm, qp_hbm,
             bu_hbm, bi_hbm, uid_hbm, iid_hbm, out_hbm,
             u2_v, i2_v, uoff_v, ioff_v, uidx_v, iidx_v,
             pd_v, qd_v, bu_v, bi_v, out_v, stage_v, sem):
    wid = lax.axis_index("s") * NUM_CORES + lax.axis_index("c")
    base = wid * ROWS_PER_WORKER

    pltpu.sync_copy(uid2_hbm.at[pl.ds(base, ROWS_PER_WORKER)], u2_v)
    pltpu.sync_copy(iid2_hbm.at[pl.ds(base, ROWS_PER_WORKER)], i2_v)
    pltpu.sync_copy(uoff_hbm.at[pl.ds(base, ROWS_PER_WORKER)], uoff_v)
    pltpu.sync_copy(ioff_hbm.at[pl.ds(base, ROWS_PER_WORKER)], ioff_v)
    pltpu.sync_copy(uid_hbm.at[pl.ds(base, ROWS_PER_WORKER)], uidx_v)
    pltpu.sync_copy(iid_hbm.at[pl.ds(base, ROWS_PER_WORKER)], iidx_v)

    cp_bu = pltpu.async_copy(bu_hbm.at[uidx_v], bu_v, sem)
    cp_bi = pltpu.async_copy(bi_hbm.at[iidx_v], bi_v, sem)
    cp_bu.wait()
    cp_bi.wait()

    lane = lax.iota(jnp.int32, N_LANES)

    for h in range(2):
        cp_p = pltpu.async_copy(
            pp_hbm.at[u2_v.at[pl.ds(h * HALF, HALF)]], pd_v, sem)
        cp_q = pltpu.async_copy(
            qp_hbm.at[i2_v.at[pl.ds(h * HALF, HALF)]], qd_v, sem)
        cp_p.wait()
        cp_q.wait()

        def group(g, _):
            rbase0 = h * HALF + g * N_LANES
            uoff_chunk = uoff_v[pl.ds(rbase0, N_LANES)]
            ioff_chunk = ioff_v[pl.ds(rbase0, N_LANES)]
            for l in range(N_LANES):
                r = g * N_LANES + l
                po = uoff_chunk[l]
                qo = ioff_chunk[l]
                v = None
                for j in range(PACKED_I32 // 16):
                    px = plsc.bitcast(pd_v[r, pl.ds(po + j * 16, 16)],
                                      jnp.bfloat16)
                    qx = plsc.bitcast(qd_v[r, pl.ds(qo + j * 16, 16)],
                                      jnp.bfloat16)
                    pa, pb = plsc.unpack(px, format=plsc.PackFormat.INTERLEAVED)
                    qa, qb = plsc.unpack(qx, format=plsc.PackFormat.INTERLEAVED)
                    t = pa * qa + pb * qb
                    v = t if v is None else v + t
                stage_v[pl.ds(l * N_LANES, N_LANES)] = v
            rbase = h * HALF + g * N_LANES
            acc = bu_v[pl.ds(rbase, N_LANES)] + bi_v[pl.ds(rbase, N_LANES)]
            for c in range(N_LANES):
                acc = acc + plsc.load_gather(stage_v, [lane * N_LANES + c])
            out_v[pl.ds(rbase, N_LANES)] = acc
            return None

        lax.fori_loop(0, GROUPS_PER_HALF, group, None)

    pltpu.sync_copy(out_v, out_hbm.at[pl.ds(base, ROWS_PER_WORKER)])


@jax.jit
def kernel(user_id, item_id, P, Q, user_bias, item_bias):
    uid = user_id.astype(jnp.int32)
    iid = item_id.astype(jnp.int32)
    mesh = plsc.VectorSubcoreMesh(
        core_axis_name="c", subcore_axis_name="s",
        num_cores=NUM_CORES, num_subcores=NUM_SUBCORES)
    run = pl.kernel(
        _mf_body,
        out_type=jax.ShapeDtypeStruct((BATCH,), jnp.float32),
        mesh=mesh,
        scratch_types=[
            pltpu.VMEM((ROWS_PER_WORKER,), jnp.int32),
            pltpu.VMEM((ROWS_PER_WORKER,), jnp.int32),
            pltpu.VMEM((ROWS_PER_WORKER,), jnp.int32),
            pltpu.VMEM((ROWS_PER_WORKER,), jnp.int32),
            pltpu.VMEM((ROWS_PER_WORKER,), jnp.int32),
            pltpu.VMEM((ROWS_PER_WORKER,), jnp.int32),
            pltpu.VMEM((HALF, PACKED_ROW_I32), jnp.int32),
            pltpu.VMEM((HALF, PACKED_ROW_I32), jnp.int32),
            pltpu.VMEM((ROWS_PER_WORKER,), jnp.float32),
            pltpu.VMEM((ROWS_PER_WORKER,), jnp.float32),
            pltpu.VMEM((ROWS_PER_WORKER,), jnp.float32),
            pltpu.VMEM((N_LANES * N_LANES,), jnp.float32),
            pltpu.SemaphoreType.DMA,
        ],
        compiler_params=pltpu.CompilerParams(
            needs_layout_passes=False, use_tc_tiling_on_sc=False),
    )
    u_hi = uid >= SPLIT
    i_hi = iid >= SPLIT
    return run(jnp.where(u_hi, uid - SPLIT, uid),
               jnp.where(i_hi, iid - SPLIT, iid),
               u_hi.astype(jnp.int32) * PACKED_I32,
               i_hi.astype(jnp.int32) * PACKED_I32,
               _repack(P.T), _repack(Q.T),
               user_bias.reshape(-1), item_bias.reshape(-1), uid, iid)


# speedup vs baseline: 2.1065x; 1.7082x over previous
import jax
import jax.numpy as jnp
from jax import lax
from jax.experimental import pallas as pl
from jax.experimental.pallas import tpu as pltpu
from jax.experimental.pallas import tpu_sc as plsc

N_LANES = 16
NUM_CORES = 2
NUM_SUBCORES = 16
NUM_WORKERS = NUM_CORES * NUM_SUBCORES
BATCH = 16384
FACTORS = 64
PACKED_W = 2 * FACTORS
ROWS_PER_WORKER = BATCH // NUM_WORKERS
HALF = ROWS_PER_WORKER // 2
GROUPS_PER_HALF = HALF // N_LANES

N_ROWS = 1000000
SPLIT = 512000
REPACK_BN = 12800
REPACK_GRID = SPLIT // REPACK_BN
REPACK_LAST_BLK = (N_ROWS - 1) // REPACK_BN


def _repack_body(lo_ref, hi_ref, out_ref):
    out_ref[:, 0:FACTORS] = lo_ref[...].T
    out_ref[:, FACTORS:PACKED_W] = hi_ref[...].T


def _repack(table_t):
    return pl.pallas_call(
        _repack_body,
        out_shape=jax.ShapeDtypeStruct((SPLIT, PACKED_W), jnp.float32),
        grid=(REPACK_GRID,),
        in_specs=[
            pl.BlockSpec((FACTORS, REPACK_BN), lambda i: (0, i)),
            pl.BlockSpec((FACTORS, REPACK_BN),
                         lambda i: (0, jnp.minimum(i + REPACK_GRID,
                                                   REPACK_LAST_BLK))),
        ],
        out_specs=pl.BlockSpec((REPACK_BN, PACKED_W), lambda i: (i, 0)),
        compiler_params=pltpu.CompilerParams(
            fuse_transposed_lhs_in_matmul=True),
    )(table_t, table_t)


def _mf_body(uid2_hbm, iid2_hbm, uoff_hbm, ioff_hbm, pp_hbm, qp_hbm,
             bu_hbm, bi_hbm, uid_hbm, iid_hbm, out_hbm,
             u2_v, i2_v, uoff_v, ioff_v, uidx_v, iidx_v,
             pd_v, qd_v, bu_v, bi_v, out_v, stage_v, sem):
    wid = lax.axis_index("s") * NUM_CORES + lax.axis_index("c")
    base = wid * ROWS_PER_WORKER

    pltpu.sync_copy(uid2_hbm.at[pl.ds(base, ROWS_PER_WORKER)], u2_v)
    pltpu.sync_copy(iid2_hbm.at[pl.ds(base, ROWS_PER_WORKER)], i2_v)
    pltpu.sync_copy(uoff_hbm.at[pl.ds(base, ROWS_PER_WORKER)], uoff_v)
    pltpu.sync_copy(ioff_hbm.at[pl.ds(base, ROWS_PER_WORKER)], ioff_v)
    pltpu.sync_copy(uid_hbm.at[pl.ds(base, ROWS_PER_WORKER)], uidx_v)
    pltpu.sync_copy(iid_hbm.at[pl.ds(base, ROWS_PER_WORKER)], iidx_v)

    cp_bu = pltpu.async_copy(bu_hbm.at[uidx_v], bu_v, sem)
    cp_bi = pltpu.async_copy(bi_hbm.at[iidx_v], bi_v, sem)
    cp_bu.wait()
    cp_bi.wait()

    lane = lax.iota(jnp.int32, N_LANES)

    for h in range(2):
        cp_p = pltpu.async_copy(
            pp_hbm.at[u2_v.at[pl.ds(h * HALF, HALF)]], pd_v, sem)
        cp_q = pltpu.async_copy(
            qp_hbm.at[i2_v.at[pl.ds(h * HALF, HALF)]], qd_v, sem)
        cp_p.wait()
        cp_q.wait()

        def group(g, _):
            rbase0 = h * HALF + g * N_LANES
            uoff_chunk = uoff_v[pl.ds(rbase0, N_LANES)]
            ioff_chunk = ioff_v[pl.ds(rbase0, N_LANES)]
            for l in range(N_LANES):
                r = g * N_LANES + l
                po = uoff_chunk[l]
                qo = ioff_chunk[l]
                v = pd_v[r, pl.ds(po, 16)] * qd_v[r, pl.ds(qo, 16)]
                for j in range(1, FACTORS // N_LANES):
                    v = v + (pd_v[r, pl.ds(po + j * 16, 16)]
                             * qd_v[r, pl.ds(qo + j * 16, 16)])
                stage_v[pl.ds(l * N_LANES, N_LANES)] = v
            rbase = h * HALF + g * N_LANES
            acc = bu_v[pl.ds(rbase, N_LANES)] + bi_v[pl.ds(rbase, N_LANES)]
            for c in range(N_LANES):
                acc = acc + plsc.load_gather(stage_v, [lane * N_LANES + c])
            out_v[pl.ds(rbase, N_LANES)] = acc
            return None

        lax.fori_loop(0, GROUPS_PER_HALF, group, None)

    pltpu.sync_copy(out_v, out_hbm.at[pl.ds(base, ROWS_PER_WORKER)])


@jax.jit
def kernel(user_id, item_id, P, Q, user_bias, item_bias):
    uid = user_id.astype(jnp.int32)
    iid = item_id.astype(jnp.int32)
    mesh = plsc.VectorSubcoreMesh(
        core_axis_name="c", subcore_axis_name="s",
        num_cores=NUM_CORES, num_subcores=NUM_SUBCORES)
    run = pl.kernel(
        _mf_body,
        out_type=jax.ShapeDtypeStruct((BATCH,), jnp.float32),
        mesh=mesh,
        scratch_types=[
            pltpu.VMEM((ROWS_PER_WORKER,), jnp.int32),
            pltpu.VMEM((ROWS_PER_WORKER,), jnp.int32),
            pltpu.VMEM((ROWS_PER_WORKER,), jnp.int32),
            pltpu.VMEM((ROWS_PER_WORKER,), jnp.int32),
            pltpu.VMEM((ROWS_PER_WORKER,), jnp.int32),
            pltpu.VMEM((ROWS_PER_WORKER,), jnp.int32),
            pltpu.VMEM((HALF, PACKED_W), jnp.float32),
            pltpu.VMEM((HALF, PACKED_W), jnp.float32),
            pltpu.VMEM((ROWS_PER_WORKER,), jnp.float32),
            pltpu.VMEM((ROWS_PER_WORKER,), jnp.float32),
            pltpu.VMEM((ROWS_PER_WORKER,), jnp.float32),
            pltpu.VMEM((N_LANES * N_LANES,), jnp.float32),
            pltpu.SemaphoreType.DMA,
        ],
        compiler_params=pltpu.CompilerParams(
            needs_layout_passes=False, use_tc_tiling_on_sc=False),
    )
    u_hi = uid >= SPLIT
    i_hi = iid >= SPLIT
    return run(jnp.where(u_hi, uid - SPLIT, uid),
               jnp.where(i_hi, iid - SPLIT, iid),
               u_hi.astype(jnp.int32) * FACTORS,
               i_hi.astype(jnp.int32) * FACTORS,
               _repack(P.T), _repack(Q.T),
               user_bias.reshape(-1), item_bias.reshape(-1), uid, iid)


# speedup vs baseline: 2.1171x; 1.0050x over previous
import jax
import jax.numpy as jnp
from jax import lax
from jax.experimental import pallas as pl
from jax.experimental.pallas import tpu as pltpu
from jax.experimental.pallas import tpu_sc as plsc

N_LANES = 16
NUM_CORES = 2
NUM_SUBCORES = 16
NUM_WORKERS = NUM_CORES * NUM_SUBCORES
BATCH = 16384
FACTORS = 64
PACKED_W = 2 * FACTORS
ROWS_PER_WORKER = BATCH // NUM_WORKERS
HALF = ROWS_PER_WORKER // 2
GROUPS_PER_HALF = HALF // N_LANES

N_ROWS = 1000000
SPLIT = 512000
REPACK_BN = 12800
REPACK_GRID = SPLIT // REPACK_BN
REPACK_LAST_BLK = (N_ROWS - 1) // REPACK_BN


def _repack_body(lo_ref, hi_ref, out_ref):
    out_ref[:, 0:FACTORS] = lo_ref[...].T
    out_ref[:, FACTORS:PACKED_W] = hi_ref[...].T


def _repack(table_t):
    return pl.pallas_call(
        _repack_body,
        out_shape=jax.ShapeDtypeStruct((SPLIT, PACKED_W), jnp.float32),
        grid=(REPACK_GRID,),
        in_specs=[
            pl.BlockSpec((FACTORS, REPACK_BN), lambda i: (0, i)),
            pl.BlockSpec((FACTORS, REPACK_BN),
                         lambda i: (0, jnp.minimum(i + REPACK_GRID,
                                                   REPACK_LAST_BLK))),
        ],
        out_specs=pl.BlockSpec((REPACK_BN, PACKED_W), lambda i: (i, 0)),
        compiler_params=pltpu.CompilerParams(
            fuse_transposed_lhs_in_matmul=True),
    )(table_t, table_t)


def _mf_body_a(uid2_hbm, uid_hbm, iid_hbm, pp_hbm, bu_hbm, bi_hbm,
               bsum_hbm, prows_hbm,
               u2_v, uidx_v, iidx_v, pd_v, bu_v, bi_v, sem):
    wid = lax.axis_index("s") * NUM_CORES + lax.axis_index("c")
    base = wid * ROWS_PER_WORKER

    pltpu.sync_copy(uid2_hbm.at[pl.ds(base, ROWS_PER_WORKER)], u2_v)
    pltpu.sync_copy(uid_hbm.at[pl.ds(base, ROWS_PER_WORKER)], uidx_v)
    pltpu.sync_copy(iid_hbm.at[pl.ds(base, ROWS_PER_WORKER)], iidx_v)

    cp_bu = pltpu.async_copy(bu_hbm.at[uidx_v], bu_v, sem)
    cp_bi = pltpu.async_copy(bi_hbm.at[iidx_v], bi_v, sem)
    cp_p = pltpu.async_copy(pp_hbm.at[u2_v], pd_v, sem)
    cp_bu.wait()
    cp_bi.wait()

    def bsum_chunk(k, _):
        bu_v[pl.ds(k * N_LANES, N_LANES)] = (
            bu_v[pl.ds(k * N_LANES, N_LANES)]
            + bi_v[pl.ds(k * N_LANES, N_LANES)])
        return None

    lax.fori_loop(0, ROWS_PER_WORKER // N_LANES, bsum_chunk, None)
    pltpu.sync_copy(bu_v, bsum_hbm.at[pl.ds(base, ROWS_PER_WORKER)])
    cp_p.wait()
    pltpu.sync_copy(pd_v, prows_hbm.at[pl.ds(base, ROWS_PER_WORKER)])


def _mf_body_b(iid2_hbm, uoff_hbm, ioff_hbm, qp_hbm, prows_hbm, bsum_hbm,
               out_hbm,
               i2_v, uoff_v, ioff_v, pd_v, qd_v, bs_v, out_v, stage_v, sem):
    wid = lax.axis_index("s") * NUM_CORES + lax.axis_index("c")
    base = wid * ROWS_PER_WORKER

    pltpu.sync_copy(iid2_hbm.at[pl.ds(base, ROWS_PER_WORKER)], i2_v)
    pltpu.sync_copy(uoff_hbm.at[pl.ds(base, ROWS_PER_WORKER)], uoff_v)
    pltpu.sync_copy(ioff_hbm.at[pl.ds(base, ROWS_PER_WORKER)], ioff_v)
    pltpu.sync_copy(bsum_hbm.at[pl.ds(base, ROWS_PER_WORKER)], bs_v)
    pltpu.sync_copy(prows_hbm.at[pl.ds(base, ROWS_PER_WORKER)], pd_v)

    lane = lax.iota(jnp.int32, N_LANES)

    for h in range(2):
        cp_q = pltpu.async_copy(
            qp_hbm.at[i2_v.at[pl.ds(h * HALF, HALF)]], qd_v, sem)
        cp_q.wait()

        def group(g, _):
            rbase = h * HALF + g * N_LANES
            uoff_chunk = uoff_v[pl.ds(rbase, N_LANES)]
            ioff_chunk = ioff_v[pl.ds(rbase, N_LANES)]
            for l in range(N_LANES):
                r = g * N_LANES + l
                po = uoff_chunk[l]
                qo = ioff_chunk[l]
                v = (pd_v[h * HALF + r, pl.ds(po, 16)]
                     * qd_v[r, pl.ds(qo, 16)])
                for j in range(1, FACTORS // N_LANES):
                    v = v + (pd_v[h * HALF + r, pl.ds(po + j * 16, 16)]
                             * qd_v[r, pl.ds(qo + j * 16, 16)])
                stage_v[pl.ds(l * N_LANES, N_LANES)] = v
            acc = bs_v[pl.ds(rbase, N_LANES)]
            for c in range(N_LANES):
                acc = acc + plsc.load_gather(stage_v, [lane * N_LANES + c])
            out_v[pl.ds(rbase, N_LANES)] = acc
            return None

        lax.fori_loop(0, GROUPS_PER_HALF, group, None)

    pltpu.sync_copy(out_v, out_hbm.at[pl.ds(base, ROWS_PER_WORKER)])


@jax.jit
def kernel(user_id, item_id, P, Q, user_bias, item_bias):
    uid = user_id.astype(jnp.int32)
    iid = item_id.astype(jnp.int32)
    mesh = plsc.VectorSubcoreMesh(
        core_axis_name="c", subcore_axis_name="s",
        num_cores=NUM_CORES, num_subcores=NUM_SUBCORES)
    run_a = pl.kernel(
        _mf_body_a,
        out_type=(jax.ShapeDtypeStruct((BATCH,), jnp.float32),
                  jax.ShapeDtypeStruct((BATCH, PACKED_W), jnp.float32)),
        mesh=mesh,
        scratch_types=[
            pltpu.VMEM((ROWS_PER_WORKER,), jnp.int32),
            pltpu.VMEM((ROWS_PER_WORKER,), jnp.int32),
            pltpu.VMEM((ROWS_PER_WORKER,), jnp.int32),
            pltpu.VMEM((ROWS_PER_WORKER, PACKED_W), jnp.float32),
            pltpu.VMEM((ROWS_PER_WORKER,), jnp.float32),
            pltpu.VMEM((ROWS_PER_WORKER,), jnp.float32),
            pltpu.SemaphoreType.DMA,
        ],
        compiler_params=pltpu.CompilerParams(
            needs_layout_passes=False, use_tc_tiling_on_sc=False),
    )
    run_b = pl.kernel(
        _mf_body_b,
        out_type=jax.ShapeDtypeStruct((BATCH,), jnp.float32),
        mesh=mesh,
        scratch_types=[
            pltpu.VMEM((ROWS_PER_WORKER,), jnp.int32),
            pltpu.VMEM((ROWS_PER_WORKER,), jnp.int32),
            pltpu.VMEM((ROWS_PER_WORKER,), jnp.int32),
            pltpu.VMEM((ROWS_PER_WORKER, PACKED_W), jnp.float32),
            pltpu.VMEM((HALF, PACKED_W), jnp.float32),
            pltpu.VMEM((ROWS_PER_WORKER,), jnp.float32),
            pltpu.VMEM((ROWS_PER_WORKER,), jnp.float32),
            pltpu.VMEM((N_LANES * N_LANES,), jnp.float32),
            pltpu.SemaphoreType.DMA,
        ],
        compiler_params=pltpu.CompilerParams(
            needs_layout_passes=False, use_tc_tiling_on_sc=False),
    )
    u_hi = uid >= SPLIT
    i_hi = iid >= SPLIT
    bsum, prows = run_a(jnp.where(u_hi, uid - SPLIT, uid), uid, iid,
                        _repack(P.T),
                        user_bias.reshape(-1), item_bias.reshape(-1))
    return run_b(jnp.where(i_hi, iid - SPLIT, iid),
                 u_hi.astype(jnp.int32) * FACTORS,
                 i_hi.astype(jnp.int32) * FACTORS,
                 _repack(Q.T), prows, bsum)
